# Initial kernel scaffold; baseline (speedup 1.0000x reference)
import jax
import jax.numpy as jnp
from jax.experimental import pallas as pl


def kernel(drdr_similarity_graph, didi_similarity_graph, drdr_dissimilarity_graph, didi_dissimilarity_graph, positive_heterograph, negative_heterograph, drug_feature, disease_feature, sample, emb_dr, emb_di, W_gt_dr, b_gt_dr, W_gt_di, b_gt_di, W_drug_lin, b_drug_lin, W_dis_lin, b_dis_lin, W_hgt, b_hgt, Wq_dr, bq_dr, Wk_dr, bk_dr, Wq_di, bq_di, Wk_di, bk_di, W1, b1, W2, b2, W3, b3, W4, b4):
    raise NotImplementedError("write your pallas kernel here")


# speedup vs baseline: 1.1913x; 1.1913x over previous
import functools

import jax
import jax.numpy as jnp
from jax import lax
from jax.experimental import pallas as pl
from jax.experimental.pallas import tpu as pltpu
from jax.experimental.pallas import tpu_sc as plsc

N_DR_ = 25000
N_DI_ = 25000
E_ = 400000
D_ = 128
H_ = 8
B_ = 16384

_EB = 128
_NBLK_REAL = E_ // _EB
_NBLK = 3200
_BPW = _NBLK // 32
_ZCH = 112


def _segsum_call(n_pad, w, n_chunks, with_deg, src3, dst3, tables):
    C = n_chunks + (1 if with_deg else 0)
    rows_per = n_pad // 16
    assert rows_per % _ZCH == 0
    mesh = plsc.VectorSubcoreMesh(core_axis_name="c", subcore_axis_name="s")

    @functools.partial(
        pl.kernel,
        mesh=mesh,
        compiler_params=pltpu.CompilerParams(use_tc_tiling_on_sc=False),
        out_type=jax.ShapeDtypeStruct((2, C, n_pad, w), jnp.float32),
        scratch_types=[
            pltpu.VMEM((_BPW, _EB), jnp.int32),
            pltpu.VMEM((_BPW, _EB), jnp.int32),
            pltpu.VMEM((_EB, w), jnp.float32),
            pltpu.VMEM((_ZCH, w), jnp.float32),
            pltpu.VMEM((_EB, w), jnp.float32),
            pltpu.VMEM_SHARED((n_pad, w), jnp.float32),
        ],
    )
    def k(src_h, dst_h, *rest):
        tabs = rest[:n_chunks]
        zrow_h = rest[n_chunks]
        ones_h = rest[n_chunks + 1]
        out_h = rest[n_chunks + 2]
        src_v, dst_v, rows_v, zbuf, obuf, acc = rest[n_chunks + 3:]
        cid = lax.axis_index("c")
        sid = lax.axis_index("s")
        wid = cid * 16 + sid
        pltpu.sync_copy(src_h.at[wid], src_v)
        pltpu.sync_copy(dst_h.at[wid], dst_v)
        pltpu.sync_copy(zrow_h, zbuf)
        pltpu.sync_copy(ones_h, obuf)
        r0 = sid * rows_per
        for c in range(C):
            @pl.loop(0, rows_per, step=_ZCH)
            def _(rz):
                pltpu.sync_copy(zbuf, acc.at[pl.ds(r0 + rz, _ZCH)])
            plsc.subcore_barrier()
            if c < n_chunks:
                @pl.loop(0, _BPW)
                def _(b):
                    pltpu.sync_copy(tabs[c].at[src_v.at[b]], rows_v)
                    pltpu.sync_copy(rows_v, acc.at[dst_v.at[b]], add=True)
            else:
                @pl.loop(0, _BPW)
                def _(b):
                    pltpu.sync_copy(obuf, acc.at[dst_v.at[b]], add=True)
            plsc.subcore_barrier()
            pltpu.sync_copy(acc.at[pl.ds(r0, rows_per)],
                            out_h.at[cid, c, pl.ds(r0, rows_per)])
            plsc.subcore_barrier()

    zrow = jnp.zeros((_ZCH, w), jnp.float32)
    ones = jnp.ones((_EB, w), jnp.float32)
    return k(src3, dst3, *tables, zrow, ones)


def _pad_edges(e, n):
    e3 = e.reshape(2, _NBLK_REAL, _EB)
    pad = jnp.full((2, _NBLK - _NBLK_REAL, _EB), n, jnp.int32)
    return jnp.concatenate([e3, pad], axis=1).reshape(2, 32, _BPW, _EB)


def _chunk_table(hw, n_pad, w):
    n = hw.shape[0]
    hwp = jnp.pad(hw, ((0, n_pad - n), (0, 0)))
    return [hwp[:, i * w:(i + 1) * w] for i in range(D_ // w)]


def _graph_pass(edges3, hw, n, n_pad, w, deg=None):
    tables = _chunk_table(hw, n_pad, w)
    with_deg = deg is None
    parts = _segsum_call(n_pad, w, len(tables), with_deg,
                         edges3[0], edges3[1], tables)
    sums = parts[0] + parts[1]
    agg = jnp.concatenate([sums[c] for c in range(len(tables))], axis=1)
    if with_deg:
        deg = sums[len(tables), :, 0]
    res = jax.nn.relu(agg / jnp.maximum(deg, 1.0)[:, None])
    return res, deg


def _self_att(x, Wq, bq, Wk, bk):
    Bn, M, Cc = x.shape
    Dh = Cc // H_
    q = (jnp.mean(x, axis=1) @ Wq + bq).reshape(Bn, 1, H_, Dh).transpose(0, 2, 1, 3)
    k = (x @ Wk + bk).reshape(Bn, M, H_, Dh).transpose(0, 2, 3, 1)
    v = x.reshape(Bn, M, H_, Dh).transpose(0, 2, 1, 3)
    alpha = jax.nn.softmax((q @ k) / (float(Dh) ** 0.5), axis=-1)
    o = alpha @ v
    return o.transpose(0, 2, 1, 3).reshape(Bn, H_ * Dh)


def _rotate(a, b):
    a_re, a_im = jnp.split(a, 2, axis=-1)
    b_re, b_im = jnp.split(b, 2, axis=-1)
    return jnp.concatenate([a_re * b_re - a_im * b_im,
                            a_re * b_im + a_im * b_re], axis=-1)


def kernel(drdr_similarity_graph, didi_similarity_graph, drdr_dissimilarity_graph, didi_dissimilarity_graph, positive_heterograph, negative_heterograph, drug_feature, disease_feature, sample, emb_dr, emb_di, W_gt_dr, b_gt_dr, W_gt_di, b_gt_di, W_drug_lin, b_drug_lin, W_dis_lin, b_dis_lin, W_hgt, b_hgt, Wq_dr, bq_dr, Wk_dr, bk_dr, Wq_di, bq_di, Wk_di, bk_di, W1, b1, W2, b2, W3, b3, W4, b4):
    n1, n1p, w1 = N_DR_, 25088, 32
    n2, n2p, w2 = N_DR_ + N_DI_, 50176, 16

    hw_dr = emb_dr @ W_gt_dr + b_gt_dr
    hw_di = emb_di @ W_gt_di + b_gt_di

    e_drdr_s = _pad_edges(drdr_similarity_graph, n1)
    e_drdr_d = _pad_edges(drdr_dissimilarity_graph, n1)
    e_didi_s = _pad_edges(didi_similarity_graph, n1)
    e_didi_d = _pad_edges(didi_dissimilarity_graph, n1)
    e_pos = _pad_edges(positive_heterograph, n2)
    e_neg = _pad_edges(negative_heterograph, n2)

    dr_sim_p, _ = _graph_pass(e_drdr_s, hw_dr, n1, n1p, w1)
    dr_sim_n, _ = _graph_pass(e_drdr_d, hw_dr, n1, n1p, w1)
    di_sim_p, _ = _graph_pass(e_didi_s, hw_di, n1, n1p, w1)
    di_sim_n, _ = _graph_pass(e_didi_d, hw_di, n1, n1p, w1)

    drug_h = drug_feature @ W_drug_lin + b_drug_lin
    dis_h = disease_feature @ W_dis_lin + b_dis_lin
    feat0 = jnp.concatenate([drug_h, dis_h], axis=0)

    fw0 = feat0 @ W_hgt + b_hgt
    f1p, deg_p = _graph_pass(e_pos, fw0, n2, n2p, w2)
    f1n, deg_n = _graph_pass(e_neg, fw0, n2, n2p, w2)
    fw1p = f1p[:n2] @ W_hgt + b_hgt
    fw1n = f1n[:n2] @ W_hgt + b_hgt
    f2p, _ = _graph_pass(e_pos, fw1p, n2, n2p, w2, deg=deg_p)
    f2n, _ = _graph_pass(e_neg, fw1n, n2, n2p, w2, deg=deg_n)

    dr = jnp.stack([dr_sim_p[:n1], dr_sim_n[:n1],
                    f2p[:N_DR_], f2n[:N_DR_]], axis=1)
    di = jnp.stack([di_sim_p[:n1], di_sim_n[:n1],
                    f2p[N_DR_:n2], f2n[N_DR_:n2]], axis=1)
    dr_final = _self_att(dr, Wq_dr, bq_dr, Wk_dr, bk_dr)
    di_final = _self_att(di, Wq_di, bq_di, Wk_di, bk_di)

    dr_s = jnp.take(dr_final, sample[:, 0], axis=0)
    di_s = jnp.take(di_final, sample[:, 1], axis=0)
    m_result = dr_s * di_s
    r_result = _rotate(dr_s, di_s)
    drdi = jnp.concatenate([dr_s, di_s, m_result, r_result], axis=1)
    h = jax.nn.relu(drdi @ W1 + b1)
    h = jax.nn.relu(h @ W2 + b2)
    h = jax.nn.relu(h @ W3 + b3)
    return h @ W4 + b4


# speedup vs baseline: 1.3762x; 1.1552x over previous
import functools

import jax
import jax.numpy as jnp
from jax import lax
from jax.experimental import pallas as pl
from jax.experimental.pallas import tpu as pltpu
from jax.experimental.pallas import tpu_sc as plsc

N_DR_ = 25000
N_DI_ = 25000
E_ = 400000
D_ = 128
H_ = 8
B_ = 16384

_EB = 128
_NBLK_REAL = E_ // _EB
_NBLK = 3200
_BPW = _NBLK // 32
_ZCH = 112
_NBUF = 4


def _segsum_call(n_pad, w, n_chunks, with_deg, src3, dst3, tables):
    C = n_chunks + (1 if with_deg else 0)
    rows_per = n_pad // 16
    assert rows_per % _ZCH == 0
    mesh = plsc.VectorSubcoreMesh(core_axis_name="c", subcore_axis_name="s")

    @functools.partial(
        pl.kernel,
        mesh=mesh,
        compiler_params=pltpu.CompilerParams(use_tc_tiling_on_sc=False),
        out_type=jax.ShapeDtypeStruct((2, C, n_pad, w), jnp.float32),
        scratch_types=[
            pltpu.VMEM((_BPW, _EB), jnp.int32),
            pltpu.VMEM((_BPW, _EB), jnp.int32),
            pltpu.VMEM((_NBUF, _EB, w), jnp.float32),
            pltpu.VMEM((_ZCH, w), jnp.float32),
            pltpu.VMEM((_EB, w), jnp.float32),
            pltpu.VMEM_SHARED((n_pad, w), jnp.float32),
            pltpu.SemaphoreType.DMA((_NBUF,)),
            pltpu.SemaphoreType.DMA((_NBUF,)),
        ],
    )
    def k(src_h, dst_h, *rest):
        tabs = rest[:n_chunks]
        zrow_h = rest[n_chunks]
        ones_h = rest[n_chunks + 1]
        out_h = rest[n_chunks + 2]
        src_v, dst_v, rows_v, zbuf, obuf, acc, gsem, ssem = rest[n_chunks + 3:]
        cid = lax.axis_index("c")
        sid = lax.axis_index("s")
        wid = cid * 16 + sid
        pltpu.sync_copy(src_h.at[wid], src_v)
        pltpu.sync_copy(dst_h.at[wid], dst_v)
        pltpu.sync_copy(zrow_h, zbuf)
        pltpu.sync_copy(ones_h, obuf)
        r0 = sid * rows_per
        for c in range(C):
            @pl.loop(0, rows_per, step=_ZCH)
            def _(rz):
                pltpu.sync_copy(zbuf, acc.at[pl.ds(r0 + rz, _ZCH)])
            plsc.subcore_barrier()
            if c < n_chunks:
                @pl.loop(0, _BPW, step=_NBUF)
                def _(g):
                    cps = [pltpu.async_copy(tabs[c].at[src_v.at[g + i]],
                                            rows_v.at[i], gsem.at[i])
                           for i in range(_NBUF)]
                    scps = []
                    for i in range(_NBUF):
                        cps[i].wait()
                        scps.append(pltpu.async_copy(
                            rows_v.at[i], acc.at[dst_v.at[g + i]],
                            ssem.at[i], add=True))
                    for i in range(_NBUF):
                        scps[i].wait()
            else:
                @pl.loop(0, _BPW, step=_NBUF)
                def _(g):
                    scps = [pltpu.async_copy(obuf, acc.at[dst_v.at[g + i]],
                                             ssem.at[i], add=True)
                            for i in range(_NBUF)]
                    for i in range(_NBUF):
                        scps[i].wait()
            plsc.subcore_barrier()
            pltpu.sync_copy(acc.at[pl.ds(r0, rows_per)],
                            out_h.at[cid, c, pl.ds(r0, rows_per)])
            plsc.subcore_barrier()

    zrow = jnp.zeros((_ZCH, w), jnp.float32)
    ones = jnp.ones((_EB, w), jnp.float32)
    return k(src3, dst3, *tables, zrow, ones)


def _pad_edges(e, n):
    e3 = e.reshape(2, _NBLK_REAL, _EB)
    pad = jnp.full((2, _NBLK - _NBLK_REAL, _EB), n, jnp.int32)
    return jnp.concatenate([e3, pad], axis=1).reshape(2, 32, _BPW, _EB)


def _chunk_table(hw, n_pad, w):
    n = hw.shape[0]
    hwp = jnp.pad(hw, ((0, n_pad - n), (0, 0)))
    return [hwp[:, i * w:(i + 1) * w] for i in range(D_ // w)]


def _graph_pass(edges3, hw, n, n_pad, w, deg=None):
    tables = _chunk_table(hw, n_pad, w)
    with_deg = deg is None
    parts = _segsum_call(n_pad, w, len(tables), with_deg,
                         edges3[0], edges3[1], tables)
    sums = parts[0] + parts[1]
    agg = jnp.concatenate([sums[c] for c in range(len(tables))], axis=1)
    if with_deg:
        deg = sums[len(tables), :, 0]
    res = jax.nn.relu(agg / jnp.maximum(deg, 1.0)[:, None])
    return res, deg


def _self_att(x, Wq, bq, Wk, bk):
    Bn, M, Cc = x.shape
    Dh = Cc // H_
    q = (jnp.mean(x, axis=1) @ Wq + bq).reshape(Bn, 1, H_, Dh).transpose(0, 2, 1, 3)
    k = (x @ Wk + bk).reshape(Bn, M, H_, Dh).transpose(0, 2, 3, 1)
    v = x.reshape(Bn, M, H_, Dh).transpose(0, 2, 1, 3)
    alpha = jax.nn.softmax((q @ k) / (float(Dh) ** 0.5), axis=-1)
    o = alpha @ v
    return o.transpose(0, 2, 1, 3).reshape(Bn, H_ * Dh)


def _rotate(a, b):
    a_re, a_im = jnp.split(a, 2, axis=-1)
    b_re, b_im = jnp.split(b, 2, axis=-1)
    return jnp.concatenate([a_re * b_re - a_im * b_im,
                            a_re * b_im + a_im * b_re], axis=-1)


def kernel(drdr_similarity_graph, didi_similarity_graph, drdr_dissimilarity_graph, didi_dissimilarity_graph, positive_heterograph, negative_heterograph, drug_feature, disease_feature, sample, emb_dr, emb_di, W_gt_dr, b_gt_dr, W_gt_di, b_gt_di, W_drug_lin, b_drug_lin, W_dis_lin, b_dis_lin, W_hgt, b_hgt, Wq_dr, bq_dr, Wk_dr, bk_dr, Wq_di, bq_di, Wk_di, bk_di, W1, b1, W2, b2, W3, b3, W4, b4):
    n1, n1p, w1 = N_DR_, 25088, 32
    n2, n2p, w2 = N_DR_ + N_DI_, 50176, 16

    hw_dr = emb_dr @ W_gt_dr + b_gt_dr
    hw_di = emb_di @ W_gt_di + b_gt_di

    e_drdr_s = _pad_edges(drdr_similarity_graph, n1)
    e_drdr_d = _pad_edges(drdr_dissimilarity_graph, n1)
    e_didi_s = _pad_edges(didi_similarity_graph, n1)
    e_didi_d = _pad_edges(didi_dissimilarity_graph, n1)
    e_pos = _pad_edges(positive_heterograph, n2)
    e_neg = _pad_edges(negative_heterograph, n2)

    dr_sim_p, _ = _graph_pass(e_drdr_s, hw_dr, n1, n1p, w1)
    dr_sim_n, _ = _graph_pass(e_drdr_d, hw_dr, n1, n1p, w1)
    di_sim_p, _ = _graph_pass(e_didi_s, hw_di, n1, n1p, w1)
    di_sim_n, _ = _graph_pass(e_didi_d, hw_di, n1, n1p, w1)

    drug_h = drug_feature @ W_drug_lin + b_drug_lin
    dis_h = disease_feature @ W_dis_lin + b_dis_lin
    feat0 = jnp.concatenate([drug_h, dis_h], axis=0)

    fw0 = feat0 @ W_hgt + b_hgt
    f1p, deg_p = _graph_pass(e_pos, fw0, n2, n2p, w2)
    f1n, deg_n = _graph_pass(e_neg, fw0, n2, n2p, w2)
    fw1p = f1p[:n2] @ W_hgt + b_hgt
    fw1n = f1n[:n2] @ W_hgt + b_hgt
    f2p, _ = _graph_pass(e_pos, fw1p, n2, n2p, w2, deg=deg_p)
    f2n, _ = _graph_pass(e_neg, fw1n, n2, n2p, w2, deg=deg_n)

    dr = jnp.stack([dr_sim_p[:n1], dr_sim_n[:n1],
                    f2p[:N_DR_], f2n[:N_DR_]], axis=1)
    di = jnp.stack([di_sim_p[:n1], di_sim_n[:n1],
                    f2p[N_DR_:n2], f2n[N_DR_:n2]], axis=1)
    dr_final = _self_att(dr, Wq_dr, bq_dr, Wk_dr, bk_dr)
    di_final = _self_att(di, Wq_di, bq_di, Wk_di, bk_di)

    dr_s = jnp.take(dr_final, sample[:, 0], axis=0)
    di_s = jnp.take(di_final, sample[:, 1], axis=0)
    m_result = dr_s * di_s
    r_result = _rotate(dr_s, di_s)
    drdi = jnp.concatenate([dr_s, di_s, m_result, r_result], axis=1)
    h = jax.nn.relu(drdi @ W1 + b1)
    h = jax.nn.relu(h @ W2 + b2)
    h = jax.nn.relu(h @ W3 + b3)
    return h @ W4 + b4
